# Initial kernel scaffold; baseline (speedup 1.0000x reference)
import jax
import jax.numpy as jnp
from jax.experimental import pallas as pl


def kernel(x, edge_index, batch, W1, b1, W2, b2, W3, b3, Wc1, bc1, Wc2, bc2):
    raise NotImplementedError("write your pallas kernel here")


# speedup vs baseline: 17.9132x; 17.9132x over previous
import functools

import jax
import jax.numpy as jnp
from jax import lax
from jax.experimental import pallas as pl
from jax.experimental.pallas import tpu as pltpu
from jax.experimental.pallas import tpu_sc as plsc

N = 10000
NPAD = 10240
E = 320000
NC = 2
NS = 16
NW = NC * NS
CH = 128
CPT = (NPAD * NW // NW) // CH
ROWS_PER_SUB = NPAD // NS
DH = 32
DEGW = 16

_mesh = plsc.VectorSubcoreMesh(core_axis_name="c", subcore_axis_name="s")
_sc_params = pltpu.CompilerParams(use_tc_tiling_on_sc=False)


@functools.partial(
    pl.kernel,
    mesh=_mesh,
    out_type=jax.ShapeDtypeStruct((NC, NPAD, DEGW), jnp.float32),
    scratch_types=[
        pltpu.VMEM((CH,), jnp.int32),
        pltpu.VMEM((CH, DEGW), jnp.float32),
        pltpu.VMEM_SHARED((NPAD, DEGW), jnp.float32),
    ],
    compiler_params=_sc_params,
)
def _sc_deg(dstw_hbm, zeros_hbm, ones_hbm, out_hbm, idx_d, ones_v, acc):
    c = lax.axis_index("c")
    s = lax.axis_index("s")
    wid = s * NC + c
    pltpu.sync_copy(zeros_hbm.at[pl.ds(s * ROWS_PER_SUB, ROWS_PER_SUB)],
                    acc.at[pl.ds(s * ROWS_PER_SUB, ROWS_PER_SUB)])
    pltpu.sync_copy(ones_hbm, ones_v)
    plsc.subcore_barrier()

    def chunk(j, carry):
        pltpu.sync_copy(dstw_hbm.at[wid, j], idx_d)
        pltpu.sync_copy(ones_v, acc.at[idx_d], add=True)
        return carry

    lax.fori_loop(0, CPT, chunk, 0)
    plsc.subcore_barrier()
    pltpu.sync_copy(acc.at[pl.ds(s * ROWS_PER_SUB, ROWS_PER_SUB)],
                    out_hbm.at[c, pl.ds(s * ROWS_PER_SUB, ROWS_PER_SUB)])


@functools.partial(
    pl.kernel,
    mesh=_mesh,
    out_type=jax.ShapeDtypeStruct((NC, NPAD, DH), jnp.float32),
    scratch_types=[
        pltpu.VMEM((CH,), jnp.int32),
        pltpu.VMEM((CH,), jnp.int32),
        pltpu.VMEM((CH, DH), jnp.float32),
        pltpu.SemaphoreType.DMA,
        pltpu.VMEM_SHARED((NPAD, DH), jnp.float32),
    ],
    compiler_params=_sc_params,
)
def _sc_layer(scaled_hbm, srcw_hbm, dstw_hbm, zeros_hbm, out_hbm,
              idx_s, idx_d, rows, sem, acc):
    c = lax.axis_index("c")
    s = lax.axis_index("s")
    wid = s * NC + c
    pltpu.sync_copy(zeros_hbm.at[pl.ds(s * ROWS_PER_SUB, ROWS_PER_SUB)],
                    acc.at[pl.ds(s * ROWS_PER_SUB, ROWS_PER_SUB)])
    plsc.subcore_barrier()

    def chunk(j, carry):
        pltpu.sync_copy(srcw_hbm.at[wid, j], idx_s)
        pltpu.async_copy(scaled_hbm.at[idx_s], rows, sem).wait()
        pltpu.sync_copy(dstw_hbm.at[wid, j], idx_d)
        pltpu.sync_copy(rows, acc.at[idx_d], add=True)
        return carry

    lax.fori_loop(0, CPT, chunk, 0)
    plsc.subcore_barrier()
    pltpu.sync_copy(acc.at[pl.ds(s * ROWS_PER_SUB, ROWS_PER_SUB)],
                    out_hbm.at[c, pl.ds(s * ROWS_PER_SUB, ROWS_PER_SUB)])


def _tc_prep_body(x_ref, w_ref, degp_ref, scaled_ref, dinv_ref):
    deg = degp_ref[0, :, 0:1] + degp_ref[1, :, 0:1] + 1.0
    dinv = lax.rsqrt(deg)
    xw = jnp.dot(x_ref[...], w_ref[...], preferred_element_type=jnp.float32)
    scaled_ref[...] = xw * dinv
    dinv_ref[...] = dinv


def _tc_prep(x_pad, W1, degp):
    return pl.pallas_call(
        _tc_prep_body,
        out_shape=[
            jax.ShapeDtypeStruct((NPAD, DH), jnp.float32),
            jax.ShapeDtypeStruct((NPAD, 1), jnp.float32),
        ],
    )(x_pad, W1, degp)


def _tc_mid_body(p_ref, sc_ref, dinv_ref, b_ref, w_ref, out_ref):
    dinv = dinv_ref[...]
    m = p_ref[0] + p_ref[1] + sc_ref[...]
    h = jnp.maximum(dinv * m + b_ref[...], 0.0)
    out_ref[...] = jnp.dot(h, w_ref[...],
                           preferred_element_type=jnp.float32) * dinv


def _tc_mid(partial, scaled, dinv, b2d, Wn):
    return pl.pallas_call(
        _tc_mid_body,
        out_shape=jax.ShapeDtypeStruct((NPAD, DH), jnp.float32),
    )(partial, scaled, dinv, b2d, Wn)


def _tc_final_body(p_ref, sc_ref, dinv_ref, b_ref, batch_ref,
                   wc1_ref, bc1_ref, wc2_ref, bc2_ref, out_ref):
    dinv = dinv_ref[...]
    m = p_ref[0] + p_ref[1] + sc_ref[...]
    h = jnp.maximum(dinv * m + b_ref[...], 0.0)
    h = h[:N]
    seg = lax.broadcasted_iota(jnp.int32, (64, N), 0)
    oneh = (seg == batch_ref[...]).astype(jnp.float32)
    cnt = jnp.sum(oneh, axis=1, keepdims=True)
    pooled = jnp.dot(oneh, h, preferred_element_type=jnp.float32)
    pooled = pooled / jnp.maximum(cnt, 1.0)
    z = jnp.maximum(
        jnp.dot(pooled, wc1_ref[...], preferred_element_type=jnp.float32)
        + bc1_ref[...], 0.0)
    out_ref[...] = jnp.dot(
        z, wc2_ref[...], preferred_element_type=jnp.float32) + bc2_ref[...]


def _tc_final(partial, scaled, dinv, b2d, batch2d, Wc1, bc1, Wc2, bc2):
    return pl.pallas_call(
        _tc_final_body,
        out_shape=jax.ShapeDtypeStruct((64, 2), jnp.float32),
    )(partial, scaled, dinv, b2d, batch2d, Wc1, bc1, Wc2, bc2)


def kernel(x, edge_index, batch, W1, b1, W2, b2, W3, b3, Wc1, bc1, Wc2, bc2):
    src = edge_index[0]
    dst = edge_index[1]
    npad_e = NW * CPT * CH - E
    pad_idx = (N + (jnp.arange(npad_e, dtype=jnp.int32) % (NPAD - N))).astype(
        jnp.int32)
    srcw = jnp.concatenate([src, pad_idx]).reshape(NW, CPT, CH)
    dstw = jnp.concatenate([dst, pad_idx]).reshape(NW, CPT, CH)

    x_pad = jnp.pad(x, ((0, NPAD - N), (0, 0)))
    zeros_acc = jnp.zeros((NPAD, DH), jnp.float32)
    zeros_deg = jnp.zeros((NPAD, DEGW), jnp.float32)
    ones_chunk = jnp.ones((CH, DEGW), jnp.float32)
    batch2d = batch.reshape(1, N)

    degp = _sc_deg(dstw, zeros_deg, ones_chunk)
    scaled1, dinv = _tc_prep(x_pad, W1, degp)
    p1 = _sc_layer(scaled1, srcw, dstw, zeros_acc)
    scaled2 = _tc_mid(p1, scaled1, dinv, b1.reshape(1, DH), W2)
    p2 = _sc_layer(scaled2, srcw, dstw, zeros_acc)
    scaled3 = _tc_mid(p2, scaled2, dinv, b2.reshape(1, DH), W3)
    p3 = _sc_layer(scaled3, srcw, dstw, zeros_acc)
    logits = _tc_final(p3, scaled3, dinv, b3.reshape(1, DH), batch2d,
                       Wc1, bc1.reshape(1, DH // 2), Wc2, bc2.reshape(1, 2))
    return logits


# speedup vs baseline: 42.5682x; 2.3764x over previous
import functools

import jax
import jax.numpy as jnp
from jax import lax
from jax.experimental import pallas as pl
from jax.experimental.pallas import tpu as pltpu
from jax.experimental.pallas import tpu_sc as plsc

N = 10000
NPAD = 10240
E = 320000
NC = 2
NS = 16
NW = NC * NS
CH = 128
CPT = (NPAD * NW // NW) // CH
ROWS_PER_SUB = NPAD // NS
DH = 32
DEGW = 16

_mesh = plsc.VectorSubcoreMesh(core_axis_name="c", subcore_axis_name="s")
_sc_params = pltpu.CompilerParams(use_tc_tiling_on_sc=False)


G = 8


@functools.partial(
    pl.kernel,
    mesh=_mesh,
    out_type=jax.ShapeDtypeStruct((NC, NPAD, DEGW), jnp.float32),
    scratch_types=[
        pltpu.VMEM((CPT, CH), jnp.int32),
        pltpu.VMEM((CH, DEGW), jnp.float32),
        pltpu.SemaphoreType.DMA,
        pltpu.VMEM_SHARED((NPAD, DEGW), jnp.float32),
    ],
    compiler_params=_sc_params,
)
def _sc_deg(dstw_hbm, zeros_hbm, ones_hbm, out_hbm, idx_d, ones_v, sem, acc):
    c = lax.axis_index("c")
    s = lax.axis_index("s")
    wid = s * NC + c
    pltpu.sync_copy(zeros_hbm.at[pl.ds(s * ROWS_PER_SUB, ROWS_PER_SUB)],
                    acc.at[pl.ds(s * ROWS_PER_SUB, ROWS_PER_SUB)])
    pltpu.sync_copy(ones_hbm, ones_v)
    pltpu.sync_copy(dstw_hbm.at[wid], idx_d)
    plsc.subcore_barrier()

    def batch(o, carry):
        handles = [
            pltpu.async_copy(ones_v, acc.at[idx_d.at[o * G + b]], sem,
                             add=True)
            for b in range(G)
        ]
        for h in handles:
            h.wait()
        return carry

    lax.fori_loop(0, CPT // G, batch, 0)
    plsc.subcore_barrier()
    pltpu.sync_copy(acc.at[pl.ds(s * ROWS_PER_SUB, ROWS_PER_SUB)],
                    out_hbm.at[c, pl.ds(s * ROWS_PER_SUB, ROWS_PER_SUB)])


@functools.partial(
    pl.kernel,
    mesh=_mesh,
    out_type=jax.ShapeDtypeStruct((NC, NPAD, DH), jnp.float32),
    scratch_types=[
        pltpu.VMEM((CPT, CH), jnp.int32),
        pltpu.VMEM((CPT, CH), jnp.int32),
        pltpu.VMEM((G, CH, DH), jnp.float32),
        pltpu.SemaphoreType.DMA,
        pltpu.SemaphoreType.DMA,
        pltpu.VMEM_SHARED((NPAD, DH), jnp.float32),
    ],
    compiler_params=_sc_params,
)
def _sc_layer(scaled_hbm, srcw_hbm, dstw_hbm, zeros_hbm, out_hbm,
              idx_s, idx_d, rows, gsem, ssem, acc):
    c = lax.axis_index("c")
    s = lax.axis_index("s")
    wid = s * NC + c
    pltpu.sync_copy(zeros_hbm.at[pl.ds(s * ROWS_PER_SUB, ROWS_PER_SUB)],
                    acc.at[pl.ds(s * ROWS_PER_SUB, ROWS_PER_SUB)])
    pltpu.sync_copy(srcw_hbm.at[wid], idx_s)
    pltpu.sync_copy(dstw_hbm.at[wid], idx_d)
    plsc.subcore_barrier()

    def batch(o, carry):
        gh = [
            pltpu.async_copy(scaled_hbm.at[idx_s.at[o * G + b]], rows.at[b],
                             gsem)
            for b in range(G)
        ]
        for h in gh:
            h.wait()
        sh = [
            pltpu.async_copy(rows.at[b], acc.at[idx_d.at[o * G + b]], ssem,
                             add=True)
            for b in range(G)
        ]
        for h in sh:
            h.wait()
        return carry

    lax.fori_loop(0, CPT // G, batch, 0)
    plsc.subcore_barrier()
    pltpu.sync_copy(acc.at[pl.ds(s * ROWS_PER_SUB, ROWS_PER_SUB)],
                    out_hbm.at[c, pl.ds(s * ROWS_PER_SUB, ROWS_PER_SUB)])


def _tc_mm_body(x_ref, w_ref, out_ref):
    out_ref[...] = jnp.dot(x_ref[...], w_ref[...],
                           preferred_element_type=jnp.float32)


def _tc_mm(x_pad, W1):
    return pl.pallas_call(
        _tc_mm_body,
        out_shape=jax.ShapeDtypeStruct((NPAD, DH), jnp.float32),
    )(x_pad, W1)


def _tc_prep_body(xw_ref, degp_ref, scaled_ref, dinv_ref):
    deg = degp_ref[0, :, 0:1] + degp_ref[1, :, 0:1] + 1.0
    dinv = lax.rsqrt(deg)
    scaled_ref[...] = xw_ref[...] * dinv
    dinv_ref[...] = dinv


def _tc_prep(xw, degp):
    return pl.pallas_call(
        _tc_prep_body,
        out_shape=[
            jax.ShapeDtypeStruct((NPAD, DH), jnp.float32),
            jax.ShapeDtypeStruct((NPAD, 1), jnp.float32),
        ],
    )(xw, degp)


def _tc_mid_body(p_ref, sc_ref, dinv_ref, b_ref, w_ref, out_ref):
    dinv = dinv_ref[...]
    m = p_ref[0] + p_ref[1] + sc_ref[...]
    h = jnp.maximum(dinv * m + b_ref[...], 0.0)
    out_ref[...] = jnp.dot(h, w_ref[...],
                           preferred_element_type=jnp.float32) * dinv


def _tc_mid(partial, scaled, dinv, b2d, Wn):
    return pl.pallas_call(
        _tc_mid_body,
        out_shape=jax.ShapeDtypeStruct((NPAD, DH), jnp.float32),
    )(partial, scaled, dinv, b2d, Wn)


def _tc_final_body(p_ref, sc_ref, dinv_ref, b_ref, batch_ref,
                   wc1_ref, bc1_ref, wc2_ref, bc2_ref, out_ref):
    dinv = dinv_ref[...]
    m = p_ref[0] + p_ref[1] + sc_ref[...]
    h = jnp.maximum(dinv * m + b_ref[...], 0.0)
    h = h[:N]
    seg = lax.broadcasted_iota(jnp.int32, (64, N), 0)
    oneh = (seg == batch_ref[...]).astype(jnp.float32)
    cnt = jnp.sum(oneh, axis=1, keepdims=True)
    pooled = jnp.dot(oneh, h, preferred_element_type=jnp.float32)
    pooled = pooled / jnp.maximum(cnt, 1.0)
    z = jnp.maximum(
        jnp.dot(pooled, wc1_ref[...], preferred_element_type=jnp.float32)
        + bc1_ref[...], 0.0)
    out_ref[...] = jnp.dot(
        z, wc2_ref[...], preferred_element_type=jnp.float32) + bc2_ref[...]


def _tc_final(partial, scaled, dinv, b2d, batch2d, Wc1, bc1, Wc2, bc2):
    return pl.pallas_call(
        _tc_final_body,
        out_shape=jax.ShapeDtypeStruct((64, 2), jnp.float32),
    )(partial, scaled, dinv, b2d, batch2d, Wc1, bc1, Wc2, bc2)


def kernel(x, edge_index, batch, W1, b1, W2, b2, W3, b3, Wc1, bc1, Wc2, bc2):
    src = edge_index[0]
    dst = edge_index[1]
    npad_e = NW * CPT * CH - E
    pad_idx = (N + (jnp.arange(npad_e, dtype=jnp.int32) % (NPAD - N))).astype(
        jnp.int32)
    srcw = jnp.concatenate([src, pad_idx]).reshape(NW, CPT, CH)
    dstw = jnp.concatenate([dst, pad_idx]).reshape(NW, CPT, CH)

    x_pad = jnp.pad(x, ((0, NPAD - N), (0, 0)))
    zeros_acc = jnp.zeros((NPAD, DH), jnp.float32)
    zeros_deg = jnp.zeros((NPAD, DEGW), jnp.float32)
    ones_chunk = jnp.ones((CH, DEGW), jnp.float32)
    batch2d = batch.reshape(1, N)

    degp = _sc_deg(dstw, zeros_deg, ones_chunk)
    xw1 = _tc_mm(x_pad, W1)
    scaled1, dinv = _tc_prep(xw1, degp)
    p1 = _sc_layer(scaled1, srcw, dstw, zeros_acc)
    scaled2 = _tc_mid(p1, scaled1, dinv, b1.reshape(1, DH), W2)
    p2 = _sc_layer(scaled2, srcw, dstw, zeros_acc)
    scaled3 = _tc_mid(p2, scaled2, dinv, b2.reshape(1, DH), W3)
    p3 = _sc_layer(scaled3, srcw, dstw, zeros_acc)
    logits = _tc_final(p3, scaled3, dinv, b3.reshape(1, DH), batch2d,
                       Wc1, bc1.reshape(1, DH // 2), Wc2, bc2.reshape(1, 2))
    return logits


# speedup vs baseline: 48.6546x; 1.1430x over previous
import functools

import jax
import jax.numpy as jnp
from jax import lax
from jax.experimental import pallas as pl
from jax.experimental.pallas import tpu as pltpu
from jax.experimental.pallas import tpu_sc as plsc

N = 10000
NPAD = 10240
E = 320000
NC = 2
NS = 16
NW = NC * NS
CH = 128
CPT = (NPAD * NW // NW) // CH
ROWS_PER_SUB = NPAD // NS
DH = 32
DEGW = 16

_mesh = plsc.VectorSubcoreMesh(core_axis_name="c", subcore_axis_name="s")
_sc_params = pltpu.CompilerParams(use_tc_tiling_on_sc=False)


G = 8


@functools.partial(
    pl.kernel,
    mesh=_mesh,
    out_type=jax.ShapeDtypeStruct((NC, NPAD, DEGW), jnp.float32),
    scratch_types=[
        pltpu.VMEM((CPT, CH), jnp.int32),
        pltpu.VMEM((CH, DEGW), jnp.float32),
        pltpu.SemaphoreType.DMA,
        pltpu.VMEM_SHARED((NPAD, DEGW), jnp.float32),
    ],
    compiler_params=_sc_params,
)
def _sc_deg(dstw_hbm, zeros_hbm, ones_hbm, out_hbm, idx_d, ones_v, sem, acc):
    c = lax.axis_index("c")
    s = lax.axis_index("s")
    wid = s * NC + c
    pltpu.sync_copy(zeros_hbm.at[pl.ds(s * ROWS_PER_SUB, ROWS_PER_SUB)],
                    acc.at[pl.ds(s * ROWS_PER_SUB, ROWS_PER_SUB)])
    pltpu.sync_copy(ones_hbm, ones_v)
    pltpu.sync_copy(dstw_hbm.at[wid], idx_d)
    plsc.subcore_barrier()

    def batch(o, carry):
        for b in range(G):
            pltpu.async_copy(ones_v, acc.at[idx_d.at[o * G + b]], sem,
                             add=True)
        return carry

    lax.fori_loop(0, CPT // G, batch, 0)

    def drain(o, carry):
        for _ in range(G):
            pltpu.make_async_copy(zeros_hbm.at[pl.ds(0, CH), pl.ds(0, DEGW)],
                                  ones_v, sem).wait()
        return carry

    lax.fori_loop(0, CPT // G, drain, 0)
    plsc.subcore_barrier()
    pltpu.sync_copy(acc.at[pl.ds(s * ROWS_PER_SUB, ROWS_PER_SUB)],
                    out_hbm.at[c, pl.ds(s * ROWS_PER_SUB, ROWS_PER_SUB)])


@functools.partial(
    pl.kernel,
    mesh=_mesh,
    out_type=jax.ShapeDtypeStruct((NC, NPAD, DH), jnp.float32),
    scratch_types=[
        pltpu.VMEM((CPT, CH), jnp.int32),
        pltpu.VMEM((CPT, CH), jnp.int32),
        pltpu.VMEM((2, G, CH, DH), jnp.float32),
        pltpu.SemaphoreType.DMA,
        pltpu.SemaphoreType.DMA,
        pltpu.VMEM_SHARED((NPAD, DH), jnp.float32),
    ],
    compiler_params=_sc_params,
)
def _sc_layer(scaled_hbm, srcw_hbm, dstw_hbm, zeros_hbm, out_hbm,
              idx_s, idx_d, rows, gsem, ssem, acc):
    c = lax.axis_index("c")
    s = lax.axis_index("s")
    wid = s * NC + c
    pltpu.sync_copy(zeros_hbm.at[pl.ds(s * ROWS_PER_SUB, ROWS_PER_SUB)],
                    acc.at[pl.ds(s * ROWS_PER_SUB, ROWS_PER_SUB)])
    pltpu.sync_copy(srcw_hbm.at[wid], idx_s)
    pltpu.sync_copy(dstw_hbm.at[wid], idx_d)
    plsc.subcore_barrier()

    NB = CPT // G

    def gathers(o, st):
        for b in range(G):
            pltpu.async_copy(scaled_hbm.at[idx_s.at[o * G + b]],
                             rows.at[st, b], gsem)

    def drain_gathers(st):
        for b in range(G):
            pltpu.make_async_copy(scaled_hbm.at[idx_s.at[b]],
                                  rows.at[st, b], gsem).wait()

    def scatters(o, st):
        for b in range(G):
            pltpu.async_copy(rows.at[st, b], acc.at[idx_d.at[o * G + b]],
                             ssem, add=True)

    def drain_scatters(st):
        for b in range(G):
            pltpu.make_async_copy(rows.at[st, b],
                                  acc.at[idx_d.at[b]], ssem).wait()

    gathers(0, 0)

    def pair(o2, carry):
        o = 2 * o2
        drain_gathers(0)
        gathers(o + 1, 1)
        scatters(o, 0)
        drain_scatters(0)
        drain_gathers(1)
        @pl.when(o2 < NB // 2 - 1)
        def _():
            gathers(o + 2, 0)
        scatters(o + 1, 1)
        drain_scatters(1)
        return carry

    lax.fori_loop(0, NB // 2, pair, 0)
    plsc.subcore_barrier()
    pltpu.sync_copy(acc.at[pl.ds(s * ROWS_PER_SUB, ROWS_PER_SUB)],
                    out_hbm.at[c, pl.ds(s * ROWS_PER_SUB, ROWS_PER_SUB)])


def _tc_mm_body(x_ref, w_ref, out_ref):
    out_ref[:N] = jnp.dot(x_ref[...], w_ref[...],
                          preferred_element_type=jnp.float32)
    out_ref[N:] = jnp.zeros((NPAD - N, DH), jnp.float32)


def _tc_mm(x, W1):
    return pl.pallas_call(
        _tc_mm_body,
        out_shape=jax.ShapeDtypeStruct((NPAD, DH), jnp.float32),
    )(x, W1)


def _tc_prep_body(xw_ref, degp_ref, scaled_ref, dinv_ref):
    deg = degp_ref[0, :, 0:1] + degp_ref[1, :, 0:1] + 1.0
    dinv = lax.rsqrt(deg)
    scaled_ref[...] = xw_ref[...] * dinv
    dinv_ref[...] = dinv


def _tc_prep(xw, degp):
    return pl.pallas_call(
        _tc_prep_body,
        out_shape=[
            jax.ShapeDtypeStruct((NPAD, DH), jnp.float32),
            jax.ShapeDtypeStruct((NPAD, 1), jnp.float32),
        ],
    )(xw, degp)


def _tc_mid_body(p_ref, sc_ref, dinv_ref, b_ref, w_ref, out_ref):
    dinv = dinv_ref[...]
    m = p_ref[0] + p_ref[1] + sc_ref[...]
    h = jnp.maximum(dinv * m + b_ref[...], 0.0)
    out_ref[...] = jnp.dot(h, w_ref[...],
                           preferred_element_type=jnp.float32) * dinv


def _tc_mid(partial, scaled, dinv, b2d, Wn):
    return pl.pallas_call(
        _tc_mid_body,
        out_shape=jax.ShapeDtypeStruct((NPAD, DH), jnp.float32),
    )(partial, scaled, dinv, b2d, Wn)


def _tc_final_body(p_ref, sc_ref, dinv_ref, b_ref, batch_ref,
                   wc1_ref, bc1_ref, wc2_ref, bc2_ref, out_ref):
    dinv = dinv_ref[...]
    m = p_ref[0] + p_ref[1] + sc_ref[...]
    h = jnp.maximum(dinv * m + b_ref[...], 0.0)
    h = h[:N]
    seg = lax.broadcasted_iota(jnp.int32, (64, N), 0)
    oneh = (seg == batch_ref[...]).astype(jnp.float32)
    cnt = jnp.sum(oneh, axis=1, keepdims=True)
    pooled = jnp.dot(oneh, h, preferred_element_type=jnp.float32)
    pooled = pooled / jnp.maximum(cnt, 1.0)
    z = jnp.maximum(
        jnp.dot(pooled, wc1_ref[...], preferred_element_type=jnp.float32)
        + bc1_ref[...], 0.0)
    out_ref[...] = jnp.dot(
        z, wc2_ref[...], preferred_element_type=jnp.float32) + bc2_ref[...]


def _tc_final(partial, scaled, dinv, b2d, batch2d, Wc1, bc1, Wc2, bc2):
    return pl.pallas_call(
        _tc_final_body,
        out_shape=jax.ShapeDtypeStruct((64, 2), jnp.float32),
    )(partial, scaled, dinv, b2d, batch2d, Wc1, bc1, Wc2, bc2)


def kernel(x, edge_index, batch, W1, b1, W2, b2, W3, b3, Wc1, bc1, Wc2, bc2):
    src = edge_index[0]
    dst = edge_index[1]
    npad_e = NW * CPT * CH - E
    pad_idx = (N + (jnp.arange(npad_e, dtype=jnp.int32) % (NPAD - N))).astype(
        jnp.int32)
    srcw = jnp.concatenate([src, pad_idx]).reshape(NW, CPT, CH)
    dstw = jnp.concatenate([dst, pad_idx]).reshape(NW, CPT, CH)

    zeros_acc = jnp.zeros((NPAD, DH), jnp.float32)
    zeros_deg = jnp.zeros((NPAD, DEGW), jnp.float32)
    ones_chunk = jnp.ones((CH, DEGW), jnp.float32)
    batch2d = batch.reshape(1, N)

    degp = _sc_deg(dstw, zeros_deg, ones_chunk)
    xw1 = _tc_mm(x, W1)
    scaled1, dinv = _tc_prep(xw1, degp)
    p1 = _sc_layer(scaled1, srcw, dstw, zeros_acc)
    scaled2 = _tc_mid(p1, scaled1, dinv, b1.reshape(1, DH), W2)
    p2 = _sc_layer(scaled2, srcw, dstw, zeros_acc)
    scaled3 = _tc_mid(p2, scaled2, dinv, b2.reshape(1, DH), W3)
    p3 = _sc_layer(scaled3, srcw, dstw, zeros_acc)
    logits = _tc_final(p3, scaled3, dinv, b3.reshape(1, DH), batch2d,
                       Wc1, bc1.reshape(1, DH // 2), Wc2, bc2.reshape(1, 2))
    return logits
